# baseline (speedup 1.0000x reference)
import functools

import jax
import jax.numpy as jnp
from jax import lax
from jax.experimental import pallas as pl
from jax.experimental.pallas import tpu as pltpu
from jax.experimental.pallas import tpu_sc as plsc

_BATCH = 16384
_EMBED_DIM = 64


@functools.lru_cache(maxsize=None)
def _make_gather_kernel(batch: int, vocab: int, dim: int):
    info = plsc.get_sparse_core_info()
    nw = info.num_cores * info.num_subcores
    ndgrp = 8
    dgrp = dim // ndgrp
    ncgrp = nw // ndgrp
    cols_per_w = batch // ncgrp
    ngrp = cols_per_w // 128
    half_cols = cols_per_w // 2
    nbuf = 4
    mesh = plsc.VectorSubcoreMesh(core_axis_name="c", subcore_axis_name="s")

    @functools.partial(
        pl.kernel,
        mesh=mesh,
        out_type=jax.ShapeDtypeStruct((dim, batch), jnp.float32),
        scratch_types=[
            pltpu.VMEM((cols_per_w,), jnp.int32),
            pltpu.VMEM((dgrp * vocab,), jnp.float32),
        ]
        + [pltpu.VMEM((8, 128), jnp.float32)] * nbuf
        + [pltpu.SemaphoreType.DMA] * (3 + nbuf),
        compiler_params=pltpu.CompilerParams(
            needs_layout_passes=False,
            disable_bounds_checks=True,
            disable_semaphore_checks=True,
            skip_device_barrier=True,
        ),
    )
    def gather_kernel(idx_hbm, table_hbm, out_hbm, idx_v, table_v, *rest):
        bufs = rest[:nbuf]
        tsem, isem0, isem1 = rest[nbuf : nbuf + 3]
        sems = rest[nbuf + 3 :]
        wid = lax.axis_index("s") * info.num_cores + lax.axis_index("c")
        g = wid % ndgrp
        c = wid // ndgrp
        d0 = g * dgrp
        base = c * cols_per_w
        tcopy = pltpu.async_copy(
            table_hbm.at[pl.ds(d0 * vocab, dgrp * vocab)], table_v, tsem
        )
        ic0 = pltpu.async_copy(
            idx_hbm.at[pl.ds(base, half_cols)], idx_v.at[pl.ds(0, half_cols)],
            isem0,
        )
        ic1 = pltpu.async_copy(
            idx_hbm.at[pl.ds(base + half_cols, half_cols)],
            idx_v.at[pl.ds(half_cols, half_cols)],
            isem1,
        )

        def fill_tile(j, buf):
            for lb in range(8):
                rows16 = idx_v[pl.ds(j * 128 + lb * 16, 16)]
                vals = [
                    plsc.load_gather(table_v, [rows16 + s * vocab])
                    for s in range(8)
                ]
                for s in range(8):
                    buf[s, pl.ds(lb * 16, 16)] = vals[s]

        def quad_body(t4, carry):
            for half in range(nbuf):
                j = nbuf * t4 + half
                buf, sem = bufs[half], sems[half]

                @pl.when(t4 > 0)
                def _():
                    pltpu.make_async_copy(
                        buf, out_hbm.at[pl.ds(0, 8), pl.ds(0, 128)], sem
                    ).wait()

                fill_tile(j, buf)
                pltpu.async_copy(
                    buf,
                    out_hbm.at[pl.ds(d0, 8), pl.ds(base + j * 128, 128)],
                    sem,
                )
            return carry

        tcopy.wait()
        ic0.wait()
        lax.fori_loop(0, ngrp // (2 * nbuf), quad_body, 0)
        ic1.wait()
        lax.fori_loop(ngrp // (2 * nbuf), ngrp // nbuf, quad_body, 0)
        for buf, sem in zip(bufs, sems):
            pltpu.make_async_copy(
                buf, out_hbm.at[pl.ds(0, 8), pl.ds(0, 128)], sem
            ).wait()

    return gather_kernel


def kernel(indices, table):
    k = _make_gather_kernel(_BATCH, table.shape[0], _EMBED_DIM)
    out_t = k(indices.astype(jnp.int32), table.T.reshape(-1))
    return out_t.T


# speedup vs baseline: 1.0864x; 1.0864x over previous
import functools

import jax
import jax.numpy as jnp
from jax import lax
from jax.experimental import pallas as pl
from jax.experimental.pallas import tpu as pltpu
from jax.experimental.pallas import tpu_sc as plsc

_BATCH = 16384
_EMBED_DIM = 64


@functools.lru_cache(maxsize=None)
def _make_gather_kernel(batch: int, vocab: int, dim: int):
    info = plsc.get_sparse_core_info()
    nw = info.num_cores * info.num_subcores
    ndgrp = 8
    dgrp = dim // ndgrp
    ncgrp = nw // ndgrp
    cols_per_w = batch // ncgrp
    ngrp = cols_per_w // 128
    mesh = plsc.VectorSubcoreMesh(core_axis_name="c", subcore_axis_name="s")

    @functools.partial(
        pl.kernel,
        mesh=mesh,
        out_type=jax.ShapeDtypeStruct((dim, batch), jnp.float32),
        scratch_types=[
            pltpu.VMEM((cols_per_w,), jnp.int32),
            pltpu.VMEM((dgrp * vocab,), jnp.float32),
            pltpu.VMEM((8, 128), jnp.float32),
            pltpu.VMEM((8, 128), jnp.float32),
            pltpu.SemaphoreType.DMA,
            pltpu.SemaphoreType.DMA,
            pltpu.SemaphoreType.DMA,
            pltpu.SemaphoreType.DMA,
        ],
        compiler_params=pltpu.CompilerParams(
            needs_layout_passes=False,
            disable_bounds_checks=True,
            disable_semaphore_checks=True,
            skip_device_barrier=True,
        ),
    )
    def gather_kernel(
        idx_hbm, table_hbm, out_hbm, idx_v, table_v,
        buf0, buf1, tsem, isem, sem0, sem1,
    ):
        wid = lax.axis_index("s") * info.num_cores + lax.axis_index("c")
        g = wid % ndgrp
        c = wid // ndgrp
        d0 = g * dgrp
        base = c * cols_per_w
        tcopy = pltpu.async_copy(
            table_hbm.at[pl.ds(d0 * vocab, dgrp * vocab)], table_v, tsem
        )
        icopy = pltpu.async_copy(idx_hbm.at[pl.ds(base, cols_per_w)], idx_v, isem)
        icopy.wait()
        tcopy.wait()

        def fill_tile(j, buf):
            for lb in range(8):
                rows16 = idx_v[pl.ds(j * 128 + lb * 16, 16)]
                vals = [
                    plsc.load_gather(table_v, [rows16 + s * vocab])
                    for s in range(8)
                ]
                for s in range(8):
                    buf[s, pl.ds(lb * 16, 16)] = vals[s]

        def pair_body(t2, carry):
            for half, buf, sem in ((0, buf0, sem0), (1, buf1, sem1)):
                j = 2 * t2 + half

                @pl.when(t2 > 0)
                def _():
                    pltpu.make_async_copy(
                        buf, out_hbm.at[pl.ds(0, 8), pl.ds(0, 128)], sem
                    ).wait()

                fill_tile(j, buf)
                pltpu.async_copy(
                    buf,
                    out_hbm.at[pl.ds(d0, 8), pl.ds(base + j * 128, 128)],
                    sem,
                )
            return carry

        lax.fori_loop(0, ngrp // 2, pair_body, 0)
        for buf, sem in ((buf0, sem0), (buf1, sem1)):
            pltpu.make_async_copy(
                buf, out_hbm.at[pl.ds(0, 8), pl.ds(0, 128)], sem
            ).wait()

    return gather_kernel


def kernel(indices, table):
    k = _make_gather_kernel(_BATCH, table.shape[0], _EMBED_DIM)
    out_t = k(indices.astype(jnp.int32), table.T.reshape(-1))
    return out_t.T
